# Initial kernel scaffold; baseline (speedup 1.0000x reference)
import jax
import jax.numpy as jnp
from jax.experimental import pallas as pl


def kernel(input, weight, bias):
    raise NotImplementedError("write your pallas kernel here")


# speedup vs baseline: 76.4087x; 76.4087x over previous
import functools
import math

import jax
import jax.numpy as jnp
from jax.experimental import pallas as pl
from jax.experimental.pallas import tpu as pltpu


def _select_body(k_active, x_ref, w_ref, b_ref, o_ref):
    x = x_ref[...]
    w = w_ref[...]
    s = jax.lax.dot_general(
        x, w, (((1,), (1,)), ((), ())), preferred_element_type=jnp.float32
    )
    bits = jax.lax.bitcast_convert_type(s, jnp.uint32)
    uk = jnp.where(
        (bits >> 31) == 0, bits | jnp.uint32(0x80000000), ~bits
    )

    def step(i, t):
        bit = jnp.uint32(31) - i.astype(jnp.uint32)
        cand = t | (jnp.uint32(1) << bit)
        cnt = jnp.sum((uk >= cand).astype(jnp.int32), axis=0, keepdims=True)
        return jnp.where(cnt >= k_active, cand, t)

    t0 = jnp.zeros((1, s.shape[1]), jnp.uint32)
    t = jax.lax.fori_loop(0, 32, step, t0)

    keep = uk >= t
    o_ref[...] = jnp.where(keep, s + b_ref[...], jnp.float32(0.0))


@jax.jit
def kernel(input, weight, bias):
    B, D = input.shape
    O = weight.shape[0]
    k_active = math.ceil(0.5 * B)
    CB = 128
    grid = (O // CB,)
    bias2 = bias.reshape(1, O)

    out = pl.pallas_call(
        functools.partial(_select_body, k_active),
        grid=grid,
        in_specs=[
            pl.BlockSpec((B, D), lambda j: (0, 0)),
            pl.BlockSpec((CB, D), lambda j: (j, 0)),
            pl.BlockSpec((1, CB), lambda j: (0, j)),
        ],
        out_specs=pl.BlockSpec((B, CB), lambda j: (0, j)),
        out_shape=jax.ShapeDtypeStruct((B, O), jnp.float32),
    )(input, weight, bias2)
    return out
